# Initial kernel scaffold; baseline (speedup 1.0000x reference)
import jax
import jax.numpy as jnp
from jax.experimental import pallas as pl


def kernel(class_embedding, entity_embeddings, Wq, bq, Wk, bk, Wv, bv, Wo, bo):
    raise NotImplementedError("write your pallas kernel here")


# speedup vs baseline: 6.5593x; 6.5593x over previous
import functools
import math

import jax
import jax.numpy as jnp
from jax import lax
from jax.experimental import pallas as pl
from jax.experimental.pallas import tpu as pltpu

_H = 4
_K = 64
_CHUNK = 8192


def _scan_kernel(a_ref, e_ref, w_out, i_out, topv_s, topi_s, m_s, z_s, *, n):
    i = pl.program_id(0)

    @pl.when(i == 0)
    def _init():
        topv_s[...] = jnp.full((_H, _K), -1e30, jnp.float32)
        topi_s[...] = jnp.zeros((_H, _K), jnp.int32)
        m_s[...] = jnp.full((_H, 128), -1e30, jnp.float32)
        z_s[...] = jnp.zeros((_H, 128), jnp.float32)

    e = e_ref[...]
    a = a_ref[...]
    s = lax.dot_general(a, e, (((1,), (1,)), ((), ())),
                        preferred_element_type=jnp.float32)
    col = lax.broadcasted_iota(jnp.int32, (_H, _CHUNK), 1)
    s = jnp.where(col < n - i * _CHUNK, s, -1e30)

    m_old = m_s[:, :1]
    z_old = z_s[:, :1]
    cmax = jnp.max(s, axis=1, keepdims=True)
    m_new = jnp.maximum(m_old, cmax)
    z_new = (z_old * jnp.exp(m_old - m_new)
             + jnp.sum(jnp.exp(s - m_new), axis=1, keepdims=True))
    m_s[...] = jnp.broadcast_to(m_new, (_H, 128))
    z_s[...] = jnp.broadcast_to(z_new, (_H, 128))

    rows = _CHUNK // 128
    s3 = s.reshape(_H, rows, 128)
    fi = (lax.broadcasted_iota(jnp.int32, (rows, 128), 0) * 128
          + lax.broadcasted_iota(jnp.int32, (rows, 128), 1))
    i64 = lax.broadcasted_iota(jnp.int32, (1, _K), 1)
    big = jnp.int32(2 ** 30)
    for h in range(_H):
        def cond(c):
            sh, tv, ti = c
            return jnp.max(sh) > jnp.min(tv)

        def body(c):
            sh, tv, ti = c
            cm = jnp.max(sh)
            p = jnp.min(jnp.where(sh == cm, fi, big))
            tmin = jnp.min(tv)
            q = jnp.min(jnp.where(tv == tmin, i64, big))
            tv = jnp.where(i64 == q, cm, tv)
            ti = jnp.where(i64 == q, p + i * _CHUNK, ti)
            sh = jnp.where(fi == p, -1e30, sh)
            return sh, tv, ti

        _, tv, ti = lax.while_loop(
            cond, body, (s3[h], topv_s[h:h + 1, :], topi_s[h:h + 1, :]))
        topv_s[h:h + 1, :] = tv
        topi_s[h:h + 1, :] = ti

    @pl.when(i == pl.num_programs(0) - 1)
    def _fin():
        w_out[...] = jnp.exp(topv_s[...] - m_s[:, :1]) / z_s[:, :1]
        i_out[...] = topi_s[...]


def _gather_kernel(idx_ref, e_ref, w_ref, wv_ref, bv_ref, wo_ref, bo_ref,
                   o_ref, acc_s, sw_s):
    g = pl.program_id(0)

    @pl.when(g == 0)
    def _init():
        acc_s[...] = jnp.zeros((_H, 32), jnp.float32)
        sw_s[...] = jnp.zeros((_H, 128), jnp.float32)

    h = g // _K
    j = lax.rem(g, _K)
    idx = idx_ref[g]
    r = lax.rem(idx, 8)
    row = e_ref[pl.ds(r, 1), :]
    wrow = w_ref[pl.ds(h, 1), :]
    i64 = lax.broadcasted_iota(jnp.int32, (1, _K), 1)
    w = jnp.sum(jnp.where(i64 == j, wrow, 0.0))
    acc_s[pl.ds(h, 1), :] = acc_s[pl.ds(h, 1), :] + w * row
    sw_s[pl.ds(h, 1), :] = sw_s[pl.ds(h, 1), :] + w

    @pl.when(g == _H * _K - 1)
    def _fin():
        acc = acc_s[...]
        sw = sw_s[:, :1]
        vsum = lax.dot_general(acc, wv_ref[...], (((1,), (1,)), ((), ())),
                               preferred_element_type=jnp.float32)
        vsum = vsum + sw * bv_ref[...]
        hd = 32 // _H
        parts = [vsum[h0:h0 + 1, hd * h0:hd * (h0 + 1)] for h0 in range(_H)]
        cc = jnp.concatenate(parts, axis=1)
        out = lax.dot_general(cc, wo_ref[...], (((1,), (1,)), ((), ())),
                              preferred_element_type=jnp.float32)
        o_ref[...] = out + bo_ref[...]


def kernel(class_embedding, entity_embeddings, Wq, bq, Wk, bk, Wv, bv, Wo, bo):
    ce = class_embedding.reshape(-1)
    q = (Wq @ ce + bq).reshape(_H, -1)
    hd = q.shape[1]
    wk3 = Wk.reshape(_H, hd, -1)
    A = jnp.einsum('hd,hdD->hD', q, wk3) / math.sqrt(hd)

    N = entity_embeddings.shape[0]
    grid_a = (N + _CHUNK - 1) // _CHUNK
    w, idx = pl.pallas_call(
        functools.partial(_scan_kernel, n=N),
        grid=(grid_a,),
        in_specs=[pl.BlockSpec((_H, 32), lambda i: (0, 0)),
                  pl.BlockSpec((_CHUNK, 32), lambda i: (i, 0))],
        out_specs=[pl.BlockSpec((_H, _K), lambda i: (0, 0)),
                   pl.BlockSpec((_H, _K), lambda i: (0, 0))],
        out_shape=[jax.ShapeDtypeStruct((_H, _K), jnp.float32),
                   jax.ShapeDtypeStruct((_H, _K), jnp.int32)],
        scratch_shapes=[pltpu.VMEM((_H, _K), jnp.float32),
                        pltpu.VMEM((_H, _K), jnp.int32),
                        pltpu.VMEM((_H, 128), jnp.float32),
                        pltpu.VMEM((_H, 128), jnp.float32)],
    )(A.astype(jnp.float32), entity_embeddings)

    flat_idx = idx.reshape(-1)
    out = pl.pallas_call(
        _gather_kernel,
        grid_spec=pltpu.PrefetchScalarGridSpec(
            num_scalar_prefetch=1,
            grid=(_H * _K,),
            in_specs=[
                pl.BlockSpec((8, 32), lambda g, idx_ref: (idx_ref[g] // 8, 0)),
                pl.BlockSpec((_H, _K), lambda g, idx_ref: (0, 0)),
                pl.BlockSpec((32, 32), lambda g, idx_ref: (0, 0)),
                pl.BlockSpec((1, 32), lambda g, idx_ref: (0, 0)),
                pl.BlockSpec((32, 32), lambda g, idx_ref: (0, 0)),
                pl.BlockSpec((1, 32), lambda g, idx_ref: (0, 0)),
            ],
            out_specs=pl.BlockSpec((1, 32), lambda g, idx_ref: (0, 0)),
            scratch_shapes=[pltpu.VMEM((_H, 32), jnp.float32),
                            pltpu.VMEM((_H, 128), jnp.float32)],
        ),
        out_shape=jax.ShapeDtypeStruct((1, 32), jnp.float32),
    )(flat_idx, entity_embeddings, w, Wv, bv.reshape(1, -1), Wo,
      bo.reshape(1, -1))
    return out.reshape(-1)


# speedup vs baseline: 7.9297x; 1.2089x over previous
import functools
import math

import jax
import jax.numpy as jnp
from jax import lax
from jax.experimental import pallas as pl
from jax.experimental.pallas import tpu as pltpu

_H = 4
_K = 64
_CHUNK = 32768


def _scan_kernel(a_ref, e_ref, w_out, i_out, topv_s, topi_s, m_s, z_s, *, n):
    i = pl.program_id(0)

    @pl.when(i == 0)
    def _init():
        topv_s[...] = jnp.full((_H, _K), -1e30, jnp.float32)
        topi_s[...] = jnp.zeros((_H, _K), jnp.int32)
        m_s[...] = jnp.full((_H, 128), -1e30, jnp.float32)
        z_s[...] = jnp.zeros((_H, 128), jnp.float32)

    e = e_ref[...]
    a = a_ref[...]
    s = lax.dot_general(a, e, (((1,), (1,)), ((), ())),
                        preferred_element_type=jnp.float32)
    col = lax.broadcasted_iota(jnp.int32, (_H, _CHUNK), 1)
    s = jnp.where(col < n - i * _CHUNK, s, -1e30)

    m_old = m_s[:, :1]
    z_old = z_s[:, :1]
    cmax = jnp.max(s, axis=1, keepdims=True)
    m_new = jnp.maximum(m_old, cmax)
    z_new = (z_old * jnp.exp(m_old - m_new)
             + jnp.sum(jnp.exp(s - m_new), axis=1, keepdims=True))
    m_s[...] = jnp.broadcast_to(m_new, (_H, 128))
    z_s[...] = jnp.broadcast_to(z_new, (_H, 128))

    rows = _CHUNK // 128
    s3 = s.reshape(_H, rows, 128)
    fi = (lax.broadcasted_iota(jnp.int32, (rows, 128), 0) * 128
          + lax.broadcasted_iota(jnp.int32, (rows, 128), 1))
    i64 = lax.broadcasted_iota(jnp.int32, (1, _K), 1)
    big = jnp.int32(2 ** 30)
    for h in range(_H):
        def cond(c):
            sh, tv, ti = c
            return jnp.max(sh) > jnp.min(tv)

        def body(c):
            sh, tv, ti = c
            cm = jnp.max(sh)
            p = jnp.min(jnp.where(sh == cm, fi, big))
            tmin = jnp.min(tv)
            q = jnp.min(jnp.where(tv == tmin, i64, big))
            tv = jnp.where(i64 == q, cm, tv)
            ti = jnp.where(i64 == q, p + i * _CHUNK, ti)
            sh = jnp.where(fi == p, -1e30, sh)
            return sh, tv, ti

        _, tv, ti = lax.while_loop(
            cond, body, (s3[h], topv_s[h:h + 1, :], topi_s[h:h + 1, :]))
        topv_s[h:h + 1, :] = tv
        topi_s[h:h + 1, :] = ti

    @pl.when(i == pl.num_programs(0) - 1)
    def _fin():
        w_out[...] = jnp.exp(topv_s[...] - m_s[:, :1]) / z_s[:, :1]
        i_out[...] = topi_s[...]


def _gather_kernel(idx_ref, e_ref, w_ref, wv_ref, bv_ref, wo_ref, bo_ref,
                   o_ref, acc_s, sw_s):
    g = pl.program_id(0)

    @pl.when(g == 0)
    def _init():
        acc_s[...] = jnp.zeros((_H, 32), jnp.float32)
        sw_s[...] = jnp.zeros((_H, 128), jnp.float32)

    h = g // _K
    j = lax.rem(g, _K)
    idx = idx_ref[g]
    r = lax.rem(idx, 8)
    row = e_ref[pl.ds(r, 1), :]
    wrow = w_ref[pl.ds(h, 1), :]
    i64 = lax.broadcasted_iota(jnp.int32, (1, _K), 1)
    w = jnp.sum(jnp.where(i64 == j, wrow, 0.0))
    acc_s[pl.ds(h, 1), :] = acc_s[pl.ds(h, 1), :] + w * row
    sw_s[pl.ds(h, 1), :] = sw_s[pl.ds(h, 1), :] + w

    @pl.when(g == _H * _K - 1)
    def _fin():
        acc = acc_s[...]
        sw = sw_s[:, :1]
        vsum = lax.dot_general(acc, wv_ref[...], (((1,), (1,)), ((), ())),
                               preferred_element_type=jnp.float32)
        vsum = vsum + sw * bv_ref[...]
        hd = 32 // _H
        parts = [vsum[h0:h0 + 1, hd * h0:hd * (h0 + 1)] for h0 in range(_H)]
        cc = jnp.concatenate(parts, axis=1)
        out = lax.dot_general(cc, wo_ref[...], (((1,), (1,)), ((), ())),
                              preferred_element_type=jnp.float32)
        o_ref[...] = out + bo_ref[...]


def kernel(class_embedding, entity_embeddings, Wq, bq, Wk, bk, Wv, bv, Wo, bo):
    ce = class_embedding.reshape(-1)
    q = (Wq @ ce + bq).reshape(_H, -1)
    hd = q.shape[1]
    wk3 = Wk.reshape(_H, hd, -1)
    A = jnp.einsum('hd,hdD->hD', q, wk3) / math.sqrt(hd)

    N = entity_embeddings.shape[0]
    grid_a = (N + _CHUNK - 1) // _CHUNK
    w, idx = pl.pallas_call(
        functools.partial(_scan_kernel, n=N),
        grid=(grid_a,),
        in_specs=[pl.BlockSpec((_H, 32), lambda i: (0, 0)),
                  pl.BlockSpec((_CHUNK, 32), lambda i: (i, 0))],
        out_specs=[pl.BlockSpec((_H, _K), lambda i: (0, 0)),
                   pl.BlockSpec((_H, _K), lambda i: (0, 0))],
        out_shape=[jax.ShapeDtypeStruct((_H, _K), jnp.float32),
                   jax.ShapeDtypeStruct((_H, _K), jnp.int32)],
        scratch_shapes=[pltpu.VMEM((_H, _K), jnp.float32),
                        pltpu.VMEM((_H, _K), jnp.int32),
                        pltpu.VMEM((_H, 128), jnp.float32),
                        pltpu.VMEM((_H, 128), jnp.float32)],
    )(A.astype(jnp.float32), entity_embeddings)

    flat_idx = idx.reshape(-1)
    out = pl.pallas_call(
        _gather_kernel,
        grid_spec=pltpu.PrefetchScalarGridSpec(
            num_scalar_prefetch=1,
            grid=(_H * _K,),
            in_specs=[
                pl.BlockSpec((8, 32), lambda g, idx_ref: (idx_ref[g] // 8, 0)),
                pl.BlockSpec((_H, _K), lambda g, idx_ref: (0, 0)),
                pl.BlockSpec((32, 32), lambda g, idx_ref: (0, 0)),
                pl.BlockSpec((1, 32), lambda g, idx_ref: (0, 0)),
                pl.BlockSpec((32, 32), lambda g, idx_ref: (0, 0)),
                pl.BlockSpec((1, 32), lambda g, idx_ref: (0, 0)),
            ],
            out_specs=pl.BlockSpec((1, 32), lambda g, idx_ref: (0, 0)),
            scratch_shapes=[pltpu.VMEM((_H, 32), jnp.float32),
                            pltpu.VMEM((_H, 128), jnp.float32)],
        ),
        out_shape=jax.ShapeDtypeStruct((1, 32), jnp.float32),
    )(flat_idx, entity_embeddings, w, Wv, bv.reshape(1, -1), Wo,
      bo.reshape(1, -1))
    return out.reshape(-1)
